# baseline (speedup 1.0000x reference)
import functools

import jax
import jax.numpy as jnp
from jax import lax
from jax.experimental import pallas as pl
from jax.experimental.pallas import tpu as pltpu
from jax.experimental.pallas import tpu_sc as plsc

B, N = 4, 4096
L = 16
NC, NS = 2, 16
NW = NC * NS
QPT = B * N // NW
TPB = NW // B
R = 2
GRP = R * L
UNROLL = 8

_MESH = plsc.VectorSubcoreMesh(core_axis_name="c", subcore_axis_name="s")


def _bf16r(v):
    u = lax.bitcast_convert_type(v, jnp.int32)
    r = (u + 0x7FFF + ((u >> 16) & 1)) & jnp.int32(-65536)
    return lax.bitcast_convert_type(r, jnp.float32)


def _prep_keys(kx_ref, ky_ref, m2x_ref, m2y_ref, b2_ref):
    def body(jc, _):
        s = jc * L
        xv = kx_ref[pl.ds(s, L)]
        yv = ky_ref[pl.ds(s, L)]
        m2x_ref[pl.ds(s, L)] = -2.0 * _bf16r(xv)
        m2y_ref[pl.ds(s, L)] = -2.0 * _bf16r(yv)
        b2_ref[pl.ds(s, L)] = xv * xv + yv * yv
        return 0

    lax.fori_loop(0, N // L, body, 0)


def _direction(qx_ref, qy_ref, m2x_ref, m2y_ref, b2_ref):
    acc = jnp.zeros((L,), jnp.float32)
    inf = jnp.full((L,), jnp.float32(jnp.inf))
    for g in range(QPT // GRP):
        base = g * GRP
        qx = [qx_ref[pl.ds(base + r * L, L)] for r in range(R)]
        qy = [qy_ref[pl.ds(base + r * L, L)] for r in range(R)]
        a2 = [qx[r] * qx[r] + qy[r] * qy[r] for r in range(R)]
        bx = [_bf16r(qx[r]) for r in range(R)]
        by = [_bf16r(qy[r]) for r in range(R)]

        def jbody(jo, carry):
            m = list(carry)
            s = jo * L
            kxv = m2x_ref[pl.ds(s, L)]
            kyv = m2y_ref[pl.ds(s, L)]
            kbv = b2_ref[pl.ds(s, L)]
            for u in range(L):
                sx = kxv[u]
                sy = kyv[u]
                sb = kbv[u]
                for r in range(R):
                    t = sy * by[r] + sb
                    t = sx * bx[r] + t
                    m[r] = jnp.minimum(m[r], t)
            return tuple(m)

        mins = lax.fori_loop(0, N // L, jbody, (inf,) * R)
        for r in range(R):
            acc = acc + jnp.maximum(a2[r] + mins[r], 0.0)
    return acc


def _body(xs1, ys1, xs2, ys2, out, qx, qy, kx, ky, m2x, m2y, b2, ob):
    wid = lax.axis_index("s") * NC + lax.axis_index("c")
    b = wid // TPB
    qoff = b * N + (wid % TPB) * QPT
    koff = b * N

    pltpu.sync_copy(xs1.at[pl.ds(qoff, QPT)], qx)
    pltpu.sync_copy(ys1.at[pl.ds(qoff, QPT)], qy)
    pltpu.sync_copy(xs2.at[pl.ds(koff, N)], kx)
    pltpu.sync_copy(ys2.at[pl.ds(koff, N)], ky)
    _prep_keys(kx, ky, m2x, m2y, b2)
    ob[...] = _direction(qx, qy, m2x, m2y, b2)
    pltpu.sync_copy(ob, out.at[pl.ds(wid * L, L)])

    pltpu.sync_copy(xs2.at[pl.ds(qoff, QPT)], qx)
    pltpu.sync_copy(ys2.at[pl.ds(qoff, QPT)], qy)
    pltpu.sync_copy(xs1.at[pl.ds(koff, N)], kx)
    pltpu.sync_copy(ys1.at[pl.ds(koff, N)], ky)
    _prep_keys(kx, ky, m2x, m2y, b2)
    ob[...] = _direction(qx, qy, m2x, m2y, b2)
    pltpu.sync_copy(ob, out.at[pl.ds(NW * L + wid * L, L)])


_sc_call = functools.partial(
    pl.kernel,
    out_type=jax.ShapeDtypeStruct((2 * NW * L,), jnp.float32),
    mesh=_MESH,
    scratch_types=[
        pltpu.VMEM((QPT,), jnp.float32),
        pltpu.VMEM((QPT,), jnp.float32),
        pltpu.VMEM((N,), jnp.float32),
        pltpu.VMEM((N,), jnp.float32),
        pltpu.VMEM((N,), jnp.float32),
        pltpu.VMEM((N,), jnp.float32),
        pltpu.VMEM((N,), jnp.float32),
        pltpu.VMEM((L,), jnp.float32),
    ],
)(_body)


@jax.jit
def kernel(points1, points2):
    xs1 = points1[..., 0].reshape(B * N)
    ys1 = points1[..., 1].reshape(B * N)
    xs2 = points2[..., 0].reshape(B * N)
    ys2 = points2[..., 1].reshape(B * N)
    partials = _sc_call(xs1, ys1, xs2, ys2)
    return jnp.sum(partials) * jnp.float32(1.0 / N)


# speedup vs baseline: 10.0980x; 10.0980x over previous
import functools

import jax
import jax.numpy as jnp
from jax import lax
from jax.experimental import pallas as pl
from jax.experimental.pallas import tpu as pltpu

B, N, M = 4, 4096, 4096
BI = 4096
NB = N // BI


def _chamfer_body(x1_ref, y1_ref, x2_ref, y2_ref, out_ref, colmin_ref):
    b = pl.program_id(0)
    ib = pl.program_id(1)

    x1 = x1_ref[0, 0, :].reshape(BI, 1)
    y1 = y1_ref[0, 0, :].reshape(BI, 1)
    x2 = x2_ref[0, 0, :].reshape(1, M)
    y2 = y2_ref[0, 0, :].reshape(1, M)

    ones = jnp.ones((BI, 1), jnp.bfloat16)
    am = jnp.concatenate(
        [
            (x1.astype(jnp.bfloat16) * jnp.bfloat16(-2.0)),
            (y1.astype(jnp.bfloat16) * jnp.bfloat16(-2.0)),
            ones,
            ones,
            ones,
        ],
        axis=1,
    )

    b2 = x2 * x2 + y2 * y2
    b2h1 = b2.astype(jnp.bfloat16)
    r1 = b2 - b2h1.astype(jnp.float32)
    b2h2 = r1.astype(jnp.bfloat16)
    b2h3 = (r1 - b2h2.astype(jnp.float32)).astype(jnp.bfloat16)
    bm = jnp.concatenate(
        [x2.astype(jnp.bfloat16), y2.astype(jnp.bfloat16), b2h1, b2h2, b2h3],
        axis=0,
    )

    q = lax.dot_general(
        am, bm, (((1,), (0,)), ((), ())),
        preferred_element_type=jnp.float32,
    )

    a2 = x1 * x1 + y1 * y1
    s = q + a2

    rowmin = jnp.min(s, axis=1)
    colmin = jnp.min(s, axis=0).reshape(1, M)

    @pl.when(ib == 0)
    def _init_col():
        colmin_ref[...] = colmin

    @pl.when(ib != 0)
    def _acc_col():
        colmin_ref[...] = jnp.minimum(colmin_ref[...], colmin)

    @pl.when(jnp.logical_and(b == 0, ib == 0))
    def _init_out():
        out_ref[0, 0] = 0.0

    partial = jnp.sum(jnp.maximum(rowmin, 0.0)) * (1.0 / N)

    @pl.when(ib == NB - 1)
    def _finish_batch():
        colsum = jnp.sum(jnp.maximum(colmin_ref[...], 0.0))
        out_ref[0, 0] += partial + colsum * (1.0 / M)

    @pl.when(ib != NB - 1)
    def _acc_row():
        out_ref[0, 0] += partial


@jax.jit
def kernel(points1, points2):
    x1 = points1[..., 0].reshape(B * NB, 1, BI)
    y1 = points1[..., 1].reshape(B * NB, 1, BI)
    x2 = points2[..., 0].reshape(B, 1, M)
    y2 = points2[..., 1].reshape(B, 1, M)

    out = pl.pallas_call(
        _chamfer_body,
        grid=(B, NB),
        in_specs=[
            pl.BlockSpec((1, 1, BI), lambda b, i: (b * NB + i, 0, 0)),
            pl.BlockSpec((1, 1, BI), lambda b, i: (b * NB + i, 0, 0)),
            pl.BlockSpec((1, 1, M), lambda b, i: (b, 0, 0)),
            pl.BlockSpec((1, 1, M), lambda b, i: (b, 0, 0)),
        ],
        out_specs=pl.BlockSpec(
            (1, 1), lambda b, i: (0, 0), memory_space=pltpu.SMEM
        ),
        out_shape=jax.ShapeDtypeStruct((1, 1), jnp.float32),
        scratch_shapes=[pltpu.VMEM((1, M), jnp.float32)],
    )(x1, y1, x2, y2)
    return out[0, 0]
